# baseline (speedup 1.0000x reference)
import functools

import jax
import jax.numpy as jnp
from jax import lax
from jax.experimental import pallas as pl
from jax.experimental.pallas import tpu as pltpu
from jax.experimental.pallas import tpu_sc as plsc

N = 4096
M = 200
DIM = 64
NW = 32
IT = 128
L = 16


def _make_sc_gather():
    mesh = plsc.VectorSubcoreMesh(core_axis_name="c", subcore_axis_name="s")

    @functools.partial(
        pl.kernel,
        mesh=mesh,
        out_type=jax.ShapeDtypeStruct((M, DIM // 8, N // IT, 8, IT), jnp.float32),
        scratch_types=[
            pltpu.VMEM((M, IT), jnp.int32),
            pltpu.VMEM((2, IT, DIM), jnp.float32),
            pltpu.VMEM((2, DIM // 8, 8, IT), jnp.float32),
            pltpu.SemaphoreType.DMA,
            pltpu.SemaphoreType.DMA,
            pltpu.SemaphoreType.DMA,
            pltpu.SemaphoreType.DMA,
        ],
        compiler_params=pltpu.CompilerParams(
            use_tc_tiling_on_sc=False, needs_layout_passes=False
        ),
    )
    def gather_kernel(xt_hbm, table_hbm, out_hbm, xblk_v, g_v, t_v,
                      gsem0, gsem1, wsem0, wsem1):
        wid = lax.axis_index("s") * 2 + lax.axis_index("c")
        gsems = (gsem0, gsem1)
        wsems = (wsem0, wsem1)

        pltpu.sync_copy(xt_hbm.at[:, pl.ds(wid * IT, IT)], xblk_v)

        def gather(j, b):
            return pltpu.make_async_copy(
                table_hbm.at[xblk_v.at[j]], g_v.at[b], gsems[b]
            )

        def write(j, b):
            return pltpu.make_async_copy(
                t_v.at[b], out_hbm.at[j, :, wid], wsems[b]
            )

        iota = lax.iota(jnp.int32, L)
        dt_c = []
        ds_c = []
        for k in range(DIM // L):
            d = iota + (L * k)
            dt_c.append(d >> 3)
            ds_c.append(d & 7)

        def transpose(b):
            def tr_body(i0, carry):
                for u in range(4):
                    i = i0 * 4 + u
                    il = jnp.full((L,), 0, jnp.int32) + i
                    for k in range(DIM // L):
                        v = g_v[b, i, pl.ds(L * k, L)]
                        plsc.store_scatter(
                            t_v.at[b], [dt_c[k], ds_c[k], il], v
                        )
                return carry

            lax.fori_loop(0, IT // 4, tr_body, 0)

        gather(0, 0).start()

        def body(jj, carry):
            for b in (0, 1):
                j = 2 * jj + b

                @pl.when(j + 1 < M)
                def _():
                    gather(j + 1, 1 - b).start()

                gather(j, b).wait()

                @pl.when(j >= 2)
                def _():
                    write(j - 2, b).wait()

                transpose(b)
                write(j, b).start()
            return carry

        lax.fori_loop(0, M // 2, body, 0)
        write(M - 2, 0).wait()
        write(M - 1, 1).wait()

    return gather_kernel


_sc_gather = _make_sc_gather()


@jax.jit
def kernel(x, E_relative_position):
    xt = x.astype(jnp.int32).T
    p = _sc_gather(xt, E_relative_position)
    return p.transpose(2, 4, 0, 1, 3).reshape(N, M, DIM)
